# baseline (speedup 1.0000x reference)
import functools

import jax
import jax.numpy as jnp
from jax import lax
from jax.experimental import pallas as pl
from jax.experimental.pallas import tpu as pltpu
from jax.experimental.pallas import tpu_sc as plsc

N = 10000
E = 320000
D = 128

NC = 2
NS = 16
NW = NC * NS
CHUNK = 80
IB = 8
CPW = 128
NBLK = CPW // IB
R = 4
E_PAD = NW * CPW * CHUNK
ROWS_PER_TILE = 640
ACC_ROWS = NS * ROWS_PER_TILE
PAD_ROW = N

_f32 = jnp.float32


def _sc_segment_sum(t, src_a, dst_a):
    mesh = plsc.VectorSubcoreMesh(core_axis_name="c", subcore_axis_name="s")

    @functools.partial(
        pl.kernel,
        out_type=jax.ShapeDtypeStruct((NC, ACC_ROWS, D), _f32),
        mesh=mesh,
        scratch_types=(
            [pltpu.VMEM((IB, CHUNK), jnp.int32)] * 4
            + [pltpu.VMEM((CHUNK, D), _f32)] * R
            + [pltpu.VMEM_SHARED((ACC_ROWS, D), _f32)]
            + [pltpu.SemaphoreType.DMA] * (2 * R + 2)
        ),
    )
    def k(t_hbm, src_hbm, dst_hbm, out_hbm, *scratch):
        srcA, dstA, srcB, dstB = scratch[:4]
        rows = scratch[4:4 + R]
        acc = scratch[4 + R]
        gsem = scratch[5 + R:5 + 2 * R]
        ssem = scratch[5 + 2 * R:5 + 3 * R]
        ip0, ip1 = scratch[5 + 3 * R:]
        r0 = rows[0]
        c = lax.axis_index("c")
        s = lax.axis_index("s")
        wid = c * NS + s

        zeros16 = jnp.zeros((16,), _f32)
        def zrow(i, carry):
            for kk in range(D // 16):
                r0[i, pl.ds(kk * 16, 16)] = zeros16
            return carry
        lax.fori_loop(0, CHUNK, zrow, 0)

        row0 = s * ROWS_PER_TILE
        def zcp(i, carry):
            pltpu.sync_copy(r0, acc.at[pl.ds(row0 + i * CHUNK, CHUNK)])
            return carry
        lax.fori_loop(0, ROWS_PER_TILE // CHUNK, zcp, 0)
        plsc.subcore_barrier()

        def gather(src_v, j, b):
            return pltpu.async_copy(t_hbm.at[src_v.at[j]], rows[b], gsem[b])

        def scatter(dst_v, j, b):
            return pltpu.async_copy(rows[b], acc.at[dst_v.at[j]], ssem[b],
                                    add=True)

        def do_block(bi, src_v, dst_v, src_n, dst_n):
            di = pltpu.async_copy(src_hbm.at[wid, pl.ds((bi + 1) * IB, IB)],
                                  src_n, ip0)
            dj = pltpu.async_copy(dst_hbm.at[wid, pl.ds((bi + 1) * IB, IB)],
                                  dst_n, ip1)
            for b in range(R):
                gather(src_v, b, b)

            def rnd(gi, carry):
                for b in range(R):
                    j = gi * R + b
                    pltpu.make_async_copy(t_hbm.at[src_v.at[j]], rows[b],
                                          gsem[b]).wait()
                    scatter(dst_v, j, b)
                for b in range(R):
                    jn = (gi + 1) * R + b
                    pltpu.make_async_copy(rows[b], acc.at[dst_v.at[jn]],
                                          ssem[b]).wait()
                    gather(src_v, jn, b)
                return carry
            lax.fori_loop(0, IB // R - 1, rnd, 0)

            for b in range(R):
                j = IB - R + b
                pltpu.make_async_copy(t_hbm.at[src_v.at[j]], rows[b],
                                      gsem[b]).wait()
                scatter(dst_v, j, b)
            for b in range(R):
                pltpu.make_async_copy(rows[b], acc.at[dst_v.at[b]],
                                      ssem[b]).wait()
            di.wait()
            dj.wait()

        pltpu.sync_copy(src_hbm.at[wid, pl.ds(0, IB)], srcA)
        pltpu.sync_copy(dst_hbm.at[wid, pl.ds(0, IB)], dstA)

        def two_blocks(h, carry):
            do_block(2 * h, srcA, dstA, srcB, dstB)
            do_block(2 * h + 1, srcB, dstB, srcA, dstA)
            return carry
        lax.fori_loop(0, NBLK // 2, two_blocks, 0)
        if NBLK % 2:
            do_block(NBLK - 1, srcA, dstA, srcB, dstB)

        plsc.subcore_barrier()
        pltpu.sync_copy(acc.at[pl.ds(row0, ROWS_PER_TILE)],
                        out_hbm.at[c, pl.ds(row0, ROWS_PER_TILE)])

    return k(t, src_a, dst_a)


def _tc_pre(symptoms, drugs, Wci_s, Wci_d, b_ci, W1s, W1d):
    R = 1000

    def body(sym, drg, wcs, wcd, bci, w1s, w1d, t1_o, causal_o):
        logit = (drg[...] @ wcd[...] + sym[...] @ wcs[...]) + bci[...]
        causal = jax.nn.sigmoid(logit)
        causal_o[...] = causal
        t1_o[...] = (sym[...] @ w1s[...] + drg[...] @ w1d[...]) * causal

    return pl.pallas_call(
        body,
        grid=(N // R,),
        in_specs=[
            pl.BlockSpec((R, 64), lambda i: (i, 0)),
            pl.BlockSpec((R, 64), lambda i: (i, 0)),
            pl.BlockSpec((64, 1), lambda i: (0, 0)),
            pl.BlockSpec((64, 1), lambda i: (0, 0)),
            pl.BlockSpec((1, 1), lambda i: (0, 0)),
            pl.BlockSpec((64, D), lambda i: (0, 0)),
            pl.BlockSpec((64, D), lambda i: (0, 0)),
        ],
        out_specs=[
            pl.BlockSpec((R, D), lambda i: (i, 0)),
            pl.BlockSpec((R, 1), lambda i: (i, 0)),
        ],
        out_shape=[
            jax.ShapeDtypeStruct((N, D), _f32),
            jax.ShapeDtypeStruct((N, 1), _f32),
        ],
    )(symptoms, drugs, Wci_s, Wci_d, b_ci, W1s, W1d)


def _tc_mid(s1p, b1, W2, causal):
    R = 1000

    def body(sp, b, w2, cz, t2_o):
        h = jnp.maximum(sp[0] + sp[1] + b[...], 0.0)
        t2_o[...] = (h @ w2[...]) * cz[...]

    return pl.pallas_call(
        body,
        grid=(N // R,),
        in_specs=[
            pl.BlockSpec((NC, R, D), lambda i: (0, i, 0)),
            pl.BlockSpec((1, D), lambda i: (0, 0)),
            pl.BlockSpec((D, D), lambda i: (0, 0)),
            pl.BlockSpec((R, 1), lambda i: (i, 0)),
        ],
        out_specs=pl.BlockSpec((R, D), lambda i: (i, 0)),
        out_shape=jax.ShapeDtypeStruct((N, D), _f32),
    )(s1p, b1, W2, causal)


def _tc_post(s2p, b2, Wp1_t, bp1_t, g_t, be_t, a_t, Wp2_t, bp2_t,
             Wp1_z, bp1_z, g_z, be_z, a_z, Wp2_z, bp2_z,
             Wc1, Wc2, bc, Wr, br, Wm, bm):
    def body(sp, b, wp1t, bp1t, gt, bet, at, wp2t, bp2t,
             wp1z, bp1z, gz, bez, az, wp2z, bp2z,
             wc1, wc2, bcz, wr, brz, wm, bmz, out_o):
        g = sp[0, :N, :] + sp[1, :N, :] + b[...]

        def proj(wp1, bp1, gg, be, a, wp2, bp2):
            h = g @ wp1[...] + bp1[...]
            mu = jnp.mean(h, axis=0, keepdims=True)
            hc = h - mu
            var = jnp.mean(hc * hc, axis=0, keepdims=True)
            hn = hc * (gg[...] / jnp.sqrt(var + 1e-5)) + be[...]
            hp = jnp.where(hn >= 0, hn, a[...] * hn)
            return hp @ wp2[...] + bp2[...]

        eA = proj(wp1t, bp1t, gt, bet, at, wp2t, bp2t)
        eB = proj(wp1z, bp1z, gz, bez, az, wp2z, bp2z)
        fused = jnp.maximum(eA @ wc1[...] + eB @ wc2[...] + bcz[...], 0.0)
        gr = jax.nn.sigmoid(fused @ wr[...] + brz[...])
        out_o[...] = gr @ wm[...] + bmz[...]

    return pl.pallas_call(
        body,
        out_shape=jax.ShapeDtypeStruct((N, 1), _f32),
    )(s2p, b2, Wp1_t, bp1_t, g_t, be_t, a_t, Wp2_t, bp2_t,
      Wp1_z, bp1_z, g_z, be_z, a_z, Wp2_z, bp2_z,
      Wc1, Wc2, bc, Wr, br, Wm, bm)


def kernel(symptoms, drugs, sigma, edge_index, W_ci, b_ci, W1, b1, W2, b2,
           Wp1_t, bp1_t, g_t, be_t, a_t, Wp2_t, bp2_t,
           Wp1_z, bp1_z, g_z, be_z, a_z, Wp2_z, bp2_z,
           Wc, bc, Wr, br, Wm, bm):
    Wci_d = (1.0 - sigma) * W_ci[:64]
    Wci_s = sigma * W_ci[64:]
    npad = E_PAD - E
    pad_src = (jnp.arange(npad, dtype=jnp.int32) * 53) % N
    pad_dst = PAD_ROW + (jnp.arange(npad, dtype=jnp.int32) % (ACC_ROWS - N))
    src = jnp.concatenate([edge_index[0], pad_src]).reshape(NW, CPW, CHUNK)
    dst = jnp.concatenate([edge_index[1], pad_dst]).reshape(NW, CPW, CHUNK)
    zblk = jnp.zeros((NW, IB, CHUNK), jnp.int32)
    src = jnp.concatenate([src, zblk], axis=1)
    dst = jnp.concatenate([dst, zblk], axis=1)

    b_ci2 = b_ci.reshape(1, 1)
    b1_2 = b1.reshape(1, D)
    b2_2 = b2.reshape(1, D)

    t1, causal = _tc_pre(symptoms, drugs, Wci_s, Wci_d, b_ci2,
                         W1[:64], W1[64:])
    s1p = _sc_segment_sum(t1, src, dst)
    t2 = _tc_mid(s1p, b1_2, W2, causal)
    s2p = _sc_segment_sum(t2, src, dst)
    return _tc_post(
        s2p, b2_2,
        Wp1_t, bp1_t.reshape(1, D), g_t.reshape(1, D), be_t.reshape(1, D),
        a_t.reshape(1, 1), Wp2_t, bp2_t.reshape(1, D),
        Wp1_z, bp1_z.reshape(1, D), g_z.reshape(1, D), be_z.reshape(1, D),
        a_z.reshape(1, 1), Wp2_z, bp2_z.reshape(1, D),
        Wc[:D], Wc[D:], bc.reshape(1, D), Wr, br.reshape(1, 1),
        Wm, bm.reshape(1, 1))


# speedup vs baseline: 1.0569x; 1.0569x over previous
import functools

import jax
import jax.numpy as jnp
from jax import lax
from jax.experimental import pallas as pl
from jax.experimental.pallas import tpu as pltpu
from jax.experimental.pallas import tpu_sc as plsc

N = 10000
E = 320000
D = 128

NC = 2
NS = 16
NW = NC * NS
CHUNK = 64
IB = 32
CPW = 160
NBLK = CPW // IB
R = 4
E_PAD = NW * CPW * CHUNK
ROWS_PER_TILE = 640
ACC_ROWS = NS * ROWS_PER_TILE
PAD_ROW = N

_f32 = jnp.float32


def _sc_segment_sum(t, src_a, dst_a):
    mesh = plsc.VectorSubcoreMesh(core_axis_name="c", subcore_axis_name="s")

    @functools.partial(
        pl.kernel,
        out_type=jax.ShapeDtypeStruct((NC, ACC_ROWS, D), _f32),
        mesh=mesh,
        scratch_types=(
            [pltpu.VMEM((IB, CHUNK), jnp.int32)] * 4
            + [pltpu.VMEM((CHUNK, D), _f32)] * R
            + [pltpu.VMEM_SHARED((ACC_ROWS, D), _f32)]
            + [pltpu.SemaphoreType.DMA] * (2 * R + 2)
        ),
    )
    def k(t_hbm, src_hbm, dst_hbm, out_hbm, *scratch):
        srcA, dstA, srcB, dstB = scratch[:4]
        rows = scratch[4:4 + R]
        acc = scratch[4 + R]
        gsem = scratch[5 + R:5 + 2 * R]
        ssem = scratch[5 + 2 * R:5 + 3 * R]
        ip0, ip1 = scratch[5 + 3 * R:]
        r0 = rows[0]
        c = lax.axis_index("c")
        s = lax.axis_index("s")
        wid = c * NS + s

        di0 = pltpu.async_copy(src_hbm.at[wid, pl.ds(0, IB)], srcA, ip0)
        dj0 = pltpu.async_copy(dst_hbm.at[wid, pl.ds(0, IB)], dstA, ip1)

        zeros16 = jnp.zeros((16,), _f32)
        def zrow(i, carry):
            for kk in range(D // 16):
                r0[i, pl.ds(kk * 16, 16)] = zeros16
            return carry
        lax.fori_loop(0, CHUNK, zrow, 0)

        row0 = s * ROWS_PER_TILE
        def zcp(i, carry):
            pltpu.sync_copy(r0, acc.at[pl.ds(row0 + i * CHUNK, CHUNK)])
            return carry
        lax.fori_loop(0, ROWS_PER_TILE // CHUNK, zcp, 0)
        di0.wait()
        dj0.wait()
        plsc.subcore_barrier()

        def gather(src_v, j, b):
            return pltpu.async_copy(t_hbm.at[src_v.at[j]], rows[b], gsem[b])

        def scatter(dst_v, j, b):
            return pltpu.async_copy(rows[b], acc.at[dst_v.at[j]], ssem[b],
                                    add=True)

        def do_block(bi, src_v, dst_v, src_n, dst_n):
            di = pltpu.async_copy(src_hbm.at[wid, pl.ds((bi + 1) * IB, IB)],
                                  src_n, ip0)
            dj = pltpu.async_copy(dst_hbm.at[wid, pl.ds((bi + 1) * IB, IB)],
                                  dst_n, ip1)

            def rnd(gi, carry):
                for b in range(R):
                    j = gi * R + b
                    pltpu.make_async_copy(t_hbm.at[src_v.at[j]], rows[b],
                                          gsem[b]).wait()
                    scatter(dst_v, j, b)
                for b in range(R):
                    jn = (gi + 1) * R + b
                    pltpu.make_async_copy(rows[b], acc.at[dst_v.at[jn]],
                                          ssem[b]).wait()
                    gather(src_v, jn, b)
                return carry
            lax.fori_loop(0, IB // R - 1, rnd, 0)

            for b in range(R):
                j = IB - R + b
                pltpu.make_async_copy(t_hbm.at[src_v.at[j]], rows[b],
                                      gsem[b]).wait()
                scatter(dst_v, j, b)
            di.wait()
            dj.wait()
            for b in range(R):
                pltpu.make_async_copy(rows[b], acc.at[dst_v.at[b]],
                                      ssem[b]).wait()
                gather(src_n, b, b)

        for b in range(R):
            gather(srcA, b, b)
        def two_blocks(h, carry):
            do_block(2 * h, srcA, dstA, srcB, dstB)
            do_block(2 * h + 1, srcB, dstB, srcA, dstA)
            return carry
        lax.fori_loop(0, NBLK // 2, two_blocks, 0)
        if NBLK % 2:
            do_block(NBLK - 1, srcA, dstA, srcB, dstB)

        for b in range(R):
            pltpu.make_async_copy(t_hbm.at[srcA.at[b]], rows[b],
                                  gsem[b]).wait()
        plsc.subcore_barrier()
        pltpu.sync_copy(acc.at[pl.ds(row0, ROWS_PER_TILE)],
                        out_hbm.at[c, pl.ds(row0, ROWS_PER_TILE)])

    return k(t, src_a, dst_a)


def _tc_pre(symptoms, drugs, Wci_s, Wci_d, b_ci, W1s, W1d):
    R = 1000

    def body(sym, drg, wcs, wcd, bci, w1s, w1d, t1_o, causal_o):
        logit = (drg[...] @ wcd[...] + sym[...] @ wcs[...]) + bci[...]
        causal = jax.nn.sigmoid(logit)
        causal_o[...] = causal
        t1_o[...] = (sym[...] @ w1s[...] + drg[...] @ w1d[...]) * causal

    return pl.pallas_call(
        body,
        grid=(N // R,),
        in_specs=[
            pl.BlockSpec((R, 64), lambda i: (i, 0)),
            pl.BlockSpec((R, 64), lambda i: (i, 0)),
            pl.BlockSpec((64, 1), lambda i: (0, 0)),
            pl.BlockSpec((64, 1), lambda i: (0, 0)),
            pl.BlockSpec((1, 1), lambda i: (0, 0)),
            pl.BlockSpec((64, D), lambda i: (0, 0)),
            pl.BlockSpec((64, D), lambda i: (0, 0)),
        ],
        out_specs=[
            pl.BlockSpec((R, D), lambda i: (i, 0)),
            pl.BlockSpec((R, 1), lambda i: (i, 0)),
        ],
        out_shape=[
            jax.ShapeDtypeStruct((N, D), _f32),
            jax.ShapeDtypeStruct((N, 1), _f32),
        ],
    )(symptoms, drugs, Wci_s, Wci_d, b_ci, W1s, W1d)


def _tc_mid(s1p, b1, W2, causal):
    R = 1000

    def body(sp, b, w2, cz, t2_o):
        h = jnp.maximum(sp[0] + sp[1] + b[...], 0.0)
        t2_o[...] = (h @ w2[...]) * cz[...]

    return pl.pallas_call(
        body,
        grid=(N // R,),
        in_specs=[
            pl.BlockSpec((NC, R, D), lambda i: (0, i, 0)),
            pl.BlockSpec((1, D), lambda i: (0, 0)),
            pl.BlockSpec((D, D), lambda i: (0, 0)),
            pl.BlockSpec((R, 1), lambda i: (i, 0)),
        ],
        out_specs=pl.BlockSpec((R, D), lambda i: (i, 0)),
        out_shape=jax.ShapeDtypeStruct((N, D), _f32),
    )(s1p, b1, W2, causal)


def _tc_post(s2p, b2, Wp1_t, bp1_t, g_t, be_t, a_t, Wp2_t, bp2_t,
             Wp1_z, bp1_z, g_z, be_z, a_z, Wp2_z, bp2_z,
             Wc1, Wc2, bc, Wr, br, Wm, bm):
    def body(sp, b, wp1t, bp1t, gt, bet, at, wp2t, bp2t,
             wp1z, bp1z, gz, bez, az, wp2z, bp2z,
             wc1, wc2, bcz, wr, brz, wm, bmz, out_o):
        g = sp[0, :N, :] + sp[1, :N, :] + b[...]

        def proj(wp1, bp1, gg, be, a, wp2, bp2):
            h = g @ wp1[...] + bp1[...]
            mu = jnp.mean(h, axis=0, keepdims=True)
            hc = h - mu
            var = jnp.mean(hc * hc, axis=0, keepdims=True)
            hn = hc * (gg[...] / jnp.sqrt(var + 1e-5)) + be[...]
            hp = jnp.where(hn >= 0, hn, a[...] * hn)
            return hp @ wp2[...] + bp2[...]

        eA = proj(wp1t, bp1t, gt, bet, at, wp2t, bp2t)
        eB = proj(wp1z, bp1z, gz, bez, az, wp2z, bp2z)
        fused = jnp.maximum(eA @ wc1[...] + eB @ wc2[...] + bcz[...], 0.0)
        gr = jax.nn.sigmoid(fused @ wr[...] + brz[...])
        out_o[...] = gr @ wm[...] + bmz[...]

    return pl.pallas_call(
        body,
        out_shape=jax.ShapeDtypeStruct((N, 1), _f32),
    )(s2p, b2, Wp1_t, bp1_t, g_t, be_t, a_t, Wp2_t, bp2_t,
      Wp1_z, bp1_z, g_z, be_z, a_z, Wp2_z, bp2_z,
      Wc1, Wc2, bc, Wr, br, Wm, bm)


def kernel(symptoms, drugs, sigma, edge_index, W_ci, b_ci, W1, b1, W2, b2,
           Wp1_t, bp1_t, g_t, be_t, a_t, Wp2_t, bp2_t,
           Wp1_z, bp1_z, g_z, be_z, a_z, Wp2_z, bp2_z,
           Wc, bc, Wr, br, Wm, bm):
    Wci_d = (1.0 - sigma) * W_ci[:64]
    Wci_s = sigma * W_ci[64:]
    npad = E_PAD - E
    pad_src = (jnp.arange(npad, dtype=jnp.int32) * 53) % N
    pad_dst = PAD_ROW + (jnp.arange(npad, dtype=jnp.int32) % (ACC_ROWS - N))
    src = jnp.concatenate([edge_index[0], pad_src]).reshape(NW, CPW, CHUNK)
    dst = jnp.concatenate([edge_index[1], pad_dst]).reshape(NW, CPW, CHUNK)
    pblk = ((jnp.arange(NW * IB * CHUNK, dtype=jnp.int32) * 37) % N
            ).reshape(NW, IB, CHUNK)
    src = jnp.concatenate([src, pblk], axis=1)
    dst = jnp.concatenate([dst, jnp.zeros((NW, IB, CHUNK), jnp.int32)], axis=1)

    b_ci2 = b_ci.reshape(1, 1)
    b1_2 = b1.reshape(1, D)
    b2_2 = b2.reshape(1, D)

    t1, causal = _tc_pre(symptoms, drugs, Wci_s, Wci_d, b_ci2,
                         W1[:64], W1[64:])
    s1p = _sc_segment_sum(t1, src, dst)
    t2 = _tc_mid(s1p, b1_2, W2, causal)
    s2p = _sc_segment_sum(t2, src, dst)
    return _tc_post(
        s2p, b2_2,
        Wp1_t, bp1_t.reshape(1, D), g_t.reshape(1, D), be_t.reshape(1, D),
        a_t.reshape(1, 1), Wp2_t, bp2_t.reshape(1, D),
        Wp1_z, bp1_z.reshape(1, D), g_z.reshape(1, D), be_z.reshape(1, D),
        a_z.reshape(1, 1), Wp2_z, bp2_z.reshape(1, D),
        Wc[:D], Wc[D:], bc.reshape(1, D), Wr, br.reshape(1, 1),
        Wm, bm.reshape(1, 1))


# speedup vs baseline: 1.0738x; 1.0160x over previous
import functools

import jax
import jax.numpy as jnp
from jax import lax
from jax.experimental import pallas as pl
from jax.experimental.pallas import tpu as pltpu
from jax.experimental.pallas import tpu_sc as plsc

N = 10000
E = 320000
D = 128

NC = 2
NS = 16
NW = NC * NS
CHUNK = 80
IB = 8
CPW = 128
NBLK = CPW // IB
R = 4
E_PAD = NW * CPW * CHUNK
ROWS_PER_TILE = 640
ACC_ROWS = NS * ROWS_PER_TILE
PAD_ROW = N

_f32 = jnp.float32


def _sc_segment_sum(t, src_a, dst_a):
    mesh = plsc.VectorSubcoreMesh(core_axis_name="c", subcore_axis_name="s")

    @functools.partial(
        pl.kernel,
        out_type=jax.ShapeDtypeStruct((NC, ACC_ROWS, D), _f32),
        mesh=mesh,
        scratch_types=(
            [pltpu.VMEM((IB, CHUNK), jnp.int32)] * 4
            + [pltpu.VMEM((CHUNK, D), _f32)] * R
            + [pltpu.VMEM_SHARED((ACC_ROWS, D), _f32)]
            + [pltpu.SemaphoreType.DMA] * (2 * R + 2)
        ),
    )
    def k(t_hbm, src_hbm, dst_hbm, out_hbm, *scratch):
        srcA, dstA, srcB, dstB = scratch[:4]
        rows = scratch[4:4 + R]
        acc = scratch[4 + R]
        gsem = scratch[5 + R:5 + 2 * R]
        ssem = scratch[5 + 2 * R:5 + 3 * R]
        ip0, ip1 = scratch[5 + 3 * R:]
        r0 = rows[0]
        c = lax.axis_index("c")
        s = lax.axis_index("s")
        wid = c * NS + s

        di0 = pltpu.async_copy(src_hbm.at[wid, pl.ds(0, IB)], srcA, ip0)
        dj0 = pltpu.async_copy(dst_hbm.at[wid, pl.ds(0, IB)], dstA, ip1)

        zeros16 = jnp.zeros((16,), _f32)
        def zrow(i, carry):
            for kk in range(D // 16):
                r0[i, pl.ds(kk * 16, 16)] = zeros16
            return carry
        lax.fori_loop(0, CHUNK, zrow, 0)

        row0 = s * ROWS_PER_TILE
        def zcp(i, carry):
            pltpu.sync_copy(r0, acc.at[pl.ds(row0 + i * CHUNK, CHUNK)])
            return carry
        lax.fori_loop(0, ROWS_PER_TILE // CHUNK, zcp, 0)
        di0.wait()
        dj0.wait()
        plsc.subcore_barrier()

        def gather(src_v, j, b):
            return pltpu.async_copy(t_hbm.at[src_v.at[j]], rows[b], gsem[b])

        def scatter(dst_v, j, b):
            return pltpu.async_copy(rows[b], acc.at[dst_v.at[j]], ssem[b],
                                    add=True)

        def do_block(bi, src_v, dst_v, src_n, dst_n):
            di = pltpu.async_copy(src_hbm.at[wid, pl.ds((bi + 1) * IB, IB)],
                                  src_n, ip0)
            dj = pltpu.async_copy(dst_hbm.at[wid, pl.ds((bi + 1) * IB, IB)],
                                  dst_n, ip1)

            def rnd(gi, carry):
                for b in range(R):
                    j = gi * R + b
                    pltpu.make_async_copy(t_hbm.at[src_v.at[j]], rows[b],
                                          gsem[b]).wait()
                    scatter(dst_v, j, b)
                for b in range(R):
                    jn = (gi + 1) * R + b
                    pltpu.make_async_copy(rows[b], acc.at[dst_v.at[jn]],
                                          ssem[b]).wait()
                    gather(src_v, jn, b)
                return carry
            lax.fori_loop(0, IB // R - 1, rnd, 0)

            for b in range(R):
                j = IB - R + b
                pltpu.make_async_copy(t_hbm.at[src_v.at[j]], rows[b],
                                      gsem[b]).wait()
                scatter(dst_v, j, b)
            di.wait()
            dj.wait()
            for b in range(R):
                pltpu.make_async_copy(rows[b], acc.at[dst_v.at[b]],
                                      ssem[b]).wait()
                gather(src_n, b, b)

        for b in range(R):
            gather(srcA, b, b)
        def two_blocks(h, carry):
            do_block(2 * h, srcA, dstA, srcB, dstB)
            do_block(2 * h + 1, srcB, dstB, srcA, dstA)
            return carry
        lax.fori_loop(0, NBLK // 2, two_blocks, 0)
        if NBLK % 2:
            do_block(NBLK - 1, srcA, dstA, srcB, dstB)

        for b in range(R):
            pltpu.make_async_copy(t_hbm.at[srcA.at[b]], rows[b],
                                  gsem[b]).wait()
        plsc.subcore_barrier()
        pltpu.sync_copy(acc.at[pl.ds(row0, ROWS_PER_TILE)],
                        out_hbm.at[c, pl.ds(row0, ROWS_PER_TILE)])

    return k(t, src_a, dst_a)


def _tc_pre(symptoms, drugs, Wci_s, Wci_d, b_ci, W1s, W1d):
    R = 1000

    def body(sym, drg, wcs, wcd, bci, w1s, w1d, t1_o, causal_o):
        logit = (drg[...] @ wcd[...] + sym[...] @ wcs[...]) + bci[...]
        causal = jax.nn.sigmoid(logit)
        causal_o[...] = causal
        t1_o[...] = (sym[...] @ w1s[...] + drg[...] @ w1d[...]) * causal

    return pl.pallas_call(
        body,
        grid=(N // R,),
        in_specs=[
            pl.BlockSpec((R, 64), lambda i: (i, 0)),
            pl.BlockSpec((R, 64), lambda i: (i, 0)),
            pl.BlockSpec((64, 1), lambda i: (0, 0)),
            pl.BlockSpec((64, 1), lambda i: (0, 0)),
            pl.BlockSpec((1, 1), lambda i: (0, 0)),
            pl.BlockSpec((64, D), lambda i: (0, 0)),
            pl.BlockSpec((64, D), lambda i: (0, 0)),
        ],
        out_specs=[
            pl.BlockSpec((R, D), lambda i: (i, 0)),
            pl.BlockSpec((R, 1), lambda i: (i, 0)),
        ],
        out_shape=[
            jax.ShapeDtypeStruct((N, D), _f32),
            jax.ShapeDtypeStruct((N, 1), _f32),
        ],
    )(symptoms, drugs, Wci_s, Wci_d, b_ci, W1s, W1d)


def _tc_mid(s1p, b1, W2, causal):
    R = 1000

    def body(sp, b, w2, cz, t2_o):
        h = jnp.maximum(sp[0] + sp[1] + b[...], 0.0)
        t2_o[...] = (h @ w2[...]) * cz[...]

    return pl.pallas_call(
        body,
        grid=(N // R,),
        in_specs=[
            pl.BlockSpec((NC, R, D), lambda i: (0, i, 0)),
            pl.BlockSpec((1, D), lambda i: (0, 0)),
            pl.BlockSpec((D, D), lambda i: (0, 0)),
            pl.BlockSpec((R, 1), lambda i: (i, 0)),
        ],
        out_specs=pl.BlockSpec((R, D), lambda i: (i, 0)),
        out_shape=jax.ShapeDtypeStruct((N, D), _f32),
    )(s1p, b1, W2, causal)


def _tc_post(s2p, b2, Wp1_t, bp1_t, g_t, be_t, a_t, Wp2_t, bp2_t,
             Wp1_z, bp1_z, g_z, be_z, a_z, Wp2_z, bp2_z,
             Wc1, Wc2, bc, Wr, br, Wm, bm):
    def body(sp, b, wp1t, bp1t, gt, bet, at, wp2t, bp2t,
             wp1z, bp1z, gz, bez, az, wp2z, bp2z,
             wc1, wc2, bcz, wr, brz, wm, bmz, out_o):
        g = sp[0, :N, :] + sp[1, :N, :] + b[...]

        def proj(wp1, bp1, gg, be, a, wp2, bp2):
            h = g @ wp1[...] + bp1[...]
            mu = jnp.mean(h, axis=0, keepdims=True)
            hc = h - mu
            var = jnp.mean(hc * hc, axis=0, keepdims=True)
            hn = hc * (gg[...] / jnp.sqrt(var + 1e-5)) + be[...]
            hp = jnp.where(hn >= 0, hn, a[...] * hn)
            return hp @ wp2[...] + bp2[...]

        eA = proj(wp1t, bp1t, gt, bet, at, wp2t, bp2t)
        eB = proj(wp1z, bp1z, gz, bez, az, wp2z, bp2z)
        fused = jnp.maximum(eA @ wc1[...] + eB @ wc2[...] + bcz[...], 0.0)
        gr = jax.nn.sigmoid(fused @ wr[...] + brz[...])
        out_o[...] = gr @ wm[...] + bmz[...]

    return pl.pallas_call(
        body,
        out_shape=jax.ShapeDtypeStruct((N, 1), _f32),
    )(s2p, b2, Wp1_t, bp1_t, g_t, be_t, a_t, Wp2_t, bp2_t,
      Wp1_z, bp1_z, g_z, be_z, a_z, Wp2_z, bp2_z,
      Wc1, Wc2, bc, Wr, br, Wm, bm)


def kernel(symptoms, drugs, sigma, edge_index, W_ci, b_ci, W1, b1, W2, b2,
           Wp1_t, bp1_t, g_t, be_t, a_t, Wp2_t, bp2_t,
           Wp1_z, bp1_z, g_z, be_z, a_z, Wp2_z, bp2_z,
           Wc, bc, Wr, br, Wm, bm):
    Wci_d = (1.0 - sigma) * W_ci[:64]
    Wci_s = sigma * W_ci[64:]
    npad = E_PAD - E
    pad_src = (jnp.arange(npad, dtype=jnp.int32) * 53) % N
    pad_dst = PAD_ROW + (jnp.arange(npad, dtype=jnp.int32) % (ACC_ROWS - N))
    src = jnp.concatenate([edge_index[0], pad_src]).reshape(NW, CPW, CHUNK)
    dst = jnp.concatenate([edge_index[1], pad_dst]).reshape(NW, CPW, CHUNK)
    pblk = ((jnp.arange(NW * IB * CHUNK, dtype=jnp.int32) * 37) % N
            ).reshape(NW, IB, CHUNK)
    src = jnp.concatenate([src, pblk], axis=1)
    dst = jnp.concatenate([dst, jnp.zeros((NW, IB, CHUNK), jnp.int32)], axis=1)

    b_ci2 = b_ci.reshape(1, 1)
    b1_2 = b1.reshape(1, D)
    b2_2 = b2.reshape(1, D)

    t1, causal = _tc_pre(symptoms, drugs, Wci_s, Wci_d, b_ci2,
                         W1[:64], W1[64:])
    s1p = _sc_segment_sum(t1, src, dst)
    t2 = _tc_mid(s1p, b1_2, W2, causal)
    s2p = _sc_segment_sum(t2, src, dst)
    return _tc_post(
        s2p, b2_2,
        Wp1_t, bp1_t.reshape(1, D), g_t.reshape(1, D), be_t.reshape(1, D),
        a_t.reshape(1, 1), Wp2_t, bp2_t.reshape(1, D),
        Wp1_z, bp1_z.reshape(1, D), g_z.reshape(1, D), be_z.reshape(1, D),
        a_z.reshape(1, 1), Wp2_z, bp2_z.reshape(1, D),
        Wc[:D], Wc[D:], bc.reshape(1, D), Wr, br.reshape(1, 1),
        Wm, bm.reshape(1, 1))
